# Initial kernel scaffold; baseline (speedup 1.0000x reference)
import jax
import jax.numpy as jnp
from jax.experimental import pallas as pl


def kernel(xyz, norm, fps_idx, knn_idx, W1, b1, W2, b2):
    raise NotImplementedError("write your pallas kernel here")


# speedup vs baseline: 35.6699x; 35.6699x over previous
import functools

import jax
import jax.numpy as jnp
from jax import lax
from jax.experimental import pallas as pl
from jax.experimental.pallas import tpu as pltpu
from jax.experimental.pallas import tpu_sc as plsc

_K = 32
_STILE = 128
_EPS = 1e-07


def _sc_gather(table, idx):
    R = idx.shape[0]
    D = table.shape[1]
    info = plsc.get_sparse_core_info()
    nw = info.num_cores * info.num_subcores
    per_w = R // nw
    n_chunks = 8
    while per_w % n_chunks or (per_w // n_chunks) % 8:
        n_chunks //= 2
    chunk = per_w // n_chunks
    mesh = plsc.VectorSubcoreMesh(core_axis_name="c", subcore_axis_name="s")

    @functools.partial(
        pl.kernel,
        mesh=mesh,
        compiler_params=pltpu.CompilerParams(use_tc_tiling_on_sc=False),
        out_type=jax.ShapeDtypeStruct((R, D), jnp.float32),
        scratch_types=[
            pltpu.VMEM((chunk,), jnp.int32),
            pltpu.VMEM((chunk, D), jnp.float32),
            pltpu.SemaphoreType.DMA,
        ],
    )
    def k(tbl_hbm, idx_hbm, out_hbm, idx_v, rows_v, sem):
        wid = lax.axis_index("s") * info.num_cores + lax.axis_index("c")
        base = wid * per_w
        for i in range(n_chunks):
            off = base + i * chunk
            pltpu.sync_copy(idx_hbm.at[pl.ds(off, chunk)], idx_v)
            pltpu.async_copy(tbl_hbm.at[idx_v], rows_v, sem).wait()
            pltpu.sync_copy(rows_v, out_hbm.at[pl.ds(off, chunk)])

    return k(table, idx)


def _dot3(ax, ay, az, bx, by, bz):
    return ax * bx + ay * by + az * bz


def _b16(x):
    return x.astype(jnp.bfloat16).astype(jnp.float32)


def _dot3_b16(ax, ay, az, bx, by, bz):
    return (_b16(ax) * _b16(bx) + _b16(ay) * _b16(by)) + _b16(az) * _b16(bz)


def _cross3(ax, ay, az, bx, by, bz):
    return (ay * bz - az * by, az * bx - ax * bz, ax * by - ay * bx)


def _unit_eps(ax, ay, az):
    ln = jnp.sqrt(ax * ax + ay * ay + az * az)
    d = ln + _EPS
    return ax / d, ay / d, az / d, ln


def _feat_kernel(g_ref, c_ref, out_ref, *, nshift):
    g = g_ref[0]
    c = c_ref[0]
    gx, gy, gz = g[0], g[1], g[2]
    gnx, gny, gnz = g[3], g[4], g[5]
    cx, cy, cz = c[0:1], c[1:2], c[2:3]
    ncx, ncy, ncz = c[3:4], c[4:5], c[5:6]

    lx, ly, lz = gx - cx, gy - cy, gz - cz
    dp = _dot3_b16(lx, ly, lz, ncx, ncy, ncz)
    px, py, pz = lx - dp * ncx, ly - dp * ncy, lz - dp * ncz
    plen = jnp.sqrt(px * px + py * py + pz * pz)
    ux, uy, uz = px / plen, py / plen, pz / plen
    ux = jnp.where(jnp.isnan(ux), 0.0, ux)
    uy = jnp.where(jnp.isnan(uy), 0.0, uy)
    uz = jnp.where(jnp.isnan(uz), 0.0, uz)

    kidx = lax.broadcasted_iota(jnp.int32, (_K, _STILE), 0)
    mval = jnp.max(plen, axis=0, keepdims=True)
    ksel = jnp.min(jnp.where(plen == mval, kidx, _K), axis=0, keepdims=True)
    selm = kidx == ksel
    vrx = jnp.sum(jnp.where(selm, ux, 0.0), axis=0, keepdims=True)
    vry = jnp.sum(jnp.where(selm, uy, 0.0), axis=0, keepdims=True)
    vrz = jnp.sum(jnp.where(selm, uz, 0.0), axis=0, keepdims=True)

    dots = _dot3_b16(ux, uy, uz, vrx, vry, vrz)
    crx, cry, crz = _cross3(ux, uy, uz, vrx, vry, vrz)
    sgn = jnp.sign(_dot3_b16(crx, cry, crz, ncx, ncy, ncz))
    sgn = jnp.where(kidx == 0, 1.0, sgn)
    d = sgn * dots - (1.0 - sgn)

    rank = jnp.zeros((_K, _STILE), jnp.int32)
    for j in range(_K):
        dj = d[j : j + 1]
        beats = (dj > d) | ((dj == d) & (j < kidx))
        rank = rank + beats.astype(jnp.int32)

    srt = [jnp.zeros((_K, _STILE), jnp.float32) for _ in range(6)]
    comps = (lx, ly, lz, gnx, gny, gnz)
    for j in range(_K):
        m = rank[j : j + 1] == kidx
        for t in range(6):
            srt[t] = jnp.where(m, comps[t][j : j + 1], srt[t])
    xix, xiy, xiz, xinx, xiny, xinz = srt

    def roll2(a, sh):
        return jnp.concatenate([a[-sh:], a[:-sh]], axis=0)

    ns = nshift
    x3x, x3y, x3z = roll2(xix, ns), roll2(xiy, ns), roll2(xiz, ns)
    x3nx, x3ny, x3nz = roll2(xinx, ns), roll2(xiny, ns), roll2(xinz, ns)
    x4x, x4y, x4z = roll2(xix, -ns), roll2(xiy, -ns), roll2(xiz, -ns)
    x4nx, x4ny, x4nz = roll2(xinx, -ns), roll2(xiny, -ns), roll2(xinz, -ns)

    uax, uay, uaz, l0 = _unit_eps(-xix, -xiy, -xiz)
    s10 = -_dot3(uax, uay, uaz, ncx, ncy, ncz)
    s20 = _dot3(uax, uay, uaz, xinx, xiny, xinz)
    ubx, uby, ubz, l1 = _unit_eps(-x3x, -x3y, -x3z)
    s11 = -_dot3(ubx, uby, ubz, ncx, ncy, ncz)
    s21 = _dot3(ubx, uby, ubz, x3nx, x3ny, x3nz)
    u12x, u12y, u12z, _ = _unit_eps(xix - x3x, xiy - x3y, xiz - x3z)
    s12 = -_dot3(u12x, u12y, u12z, xinx, xiny, xinz)
    s22 = _dot3(u12x, u12y, u12z, x3nx, x3ny, x3nz)
    angle_0 = _dot3(uax, uay, uaz, ubx, uby, ubz)
    angle_1 = _dot3(-ubx, -uby, -ubz, -u12x, -u12y, -u12z)

    pxx, pxy, pxz, _ = _unit_eps(x4x, x4y, x4z)
    p2x, p2y, p2z, _ = _unit_eps(xix, xiy, xiz)
    xxx, xxy, xxz, _ = _unit_eps(x4x - xix, x4y - xiy, x4z - xiz)
    p3x, p3y, p3z, _ = _unit_eps(x3x, x3y, x3z)
    sn1x, sn1y, sn1z = _cross3(pxx, pxy, pxz, p2x, p2y, p2z)
    sn2x, sn2y, sn2z = _cross3(p3x, p3y, p3z, p2x, p2y, p2z)
    a11 = _dot3(pxx, pxy, pxz, p2x, p2y, p2z)
    a12 = _dot3(pxx, pxy, pxz, xxx, xxy, xxz)
    a3 = _dot3(sn1x, sn1y, sn1z, sn2x, sn2y, sn2z)
    a21 = _dot3(xxx, xxy, xxz, x4nx, x4ny, x4nz)
    a22 = _dot3(p2x, p2y, p2z, x4nx, x4ny, x4nz)

    out_ref[0] = jnp.stack(
        [l0, s10, s20, angle_0, s11, s21, angle_1, s12, s22,
         a11, a12, a21, a22, a3],
        axis=0,
    )


def _mlp_kernel(rf_ref, w1_ref, b1_ref, w2_ref, b2_ref, out_ref):
    x = rf_ref[0]
    xm = x.reshape(_STILE * _K, 14)
    h = jnp.maximum(
        jnp.dot(xm, w1_ref[...], preferred_element_type=jnp.float32)
        + b1_ref[...], 0.0)
    h = jnp.maximum(
        jnp.dot(h, w2_ref[...], preferred_element_type=jnp.float32)
        + b2_ref[...], 0.0)
    out_ref[0] = jnp.max(h.reshape(_STILE, _K, 64), axis=1)


def kernel(xyz, norm, fps_idx, knn_idx, W1, b1, W2, b2):
    B, N, _ = xyz.shape
    S = fps_idx.shape[1]
    K = knn_idx.shape[2]

    table = jnp.concatenate(
        [xyz, norm, jnp.zeros((B, N, 2), jnp.float32)], axis=-1
    ).reshape(B * N, 8)
    offs = (jnp.arange(B, dtype=jnp.int32) * N)
    idx_knn = (knn_idx.astype(jnp.int32) + offs[:, None, None]).reshape(-1)
    idx_fps = (fps_idx.astype(jnp.int32) + offs[:, None]).reshape(-1)
    idx_all = jnp.concatenate([idx_knn, idx_fps])
    rows = _sc_gather(table, idx_all)
    grouped = rows[: B * S * K].reshape(B, S, K, 8)
    centres = rows[B * S * K :].reshape(B, S, 8)
    new_xyz = centres[..., 0:3]
    new_norm = centres[..., 3:6]

    G = grouped.transpose(0, 3, 2, 1)
    C = centres.transpose(0, 2, 1)
    rf = pl.pallas_call(
        functools.partial(_feat_kernel, nshift=2 if S >= 1024 else 1),
        grid=(B, S // _STILE),
        in_specs=[
            pl.BlockSpec((1, 8, K, _STILE), lambda b, s: (b, 0, 0, s)),
            pl.BlockSpec((1, 8, _STILE), lambda b, s: (b, 0, s)),
        ],
        out_specs=pl.BlockSpec((1, 14, K, _STILE), lambda b, s: (b, 0, 0, s)),
        out_shape=jax.ShapeDtypeStruct((B, 14, K, S), jnp.float32),
    )(G, C)

    rft = rf.transpose(0, 3, 2, 1)
    new_points = pl.pallas_call(
        _mlp_kernel,
        grid=(B, S // _STILE),
        in_specs=[
            pl.BlockSpec((1, _STILE, K, 14), lambda b, s: (b, s, 0, 0)),
            pl.BlockSpec((14, 32), lambda b, s: (0, 0)),
            pl.BlockSpec((1, 32), lambda b, s: (0, 0)),
            pl.BlockSpec((32, 64), lambda b, s: (0, 0)),
            pl.BlockSpec((1, 64), lambda b, s: (0, 0)),
        ],
        out_specs=pl.BlockSpec((1, _STILE, 64), lambda b, s: (b, s, 0)),
        out_shape=jax.ShapeDtypeStruct((B, S, 64), jnp.float32),
    )(rft, W1, b1.reshape(1, 32), W2, b2.reshape(1, 64))

    return new_xyz, new_norm, new_points


# speedup vs baseline: 48.7708x; 1.3673x over previous
import functools

import jax
import jax.numpy as jnp
from jax import lax
from jax.experimental import pallas as pl
from jax.experimental.pallas import tpu as pltpu
from jax.experimental.pallas import tpu_sc as plsc

_K = 32
_STILE = 128
_EPS = 1e-07


def _sc_gather(table, idx):
    R = idx.shape[0]
    D = table.shape[1]
    info = plsc.get_sparse_core_info()
    nw = info.num_cores * info.num_subcores
    per_w = R // nw
    n_chunks = 8
    while per_w % n_chunks or (per_w // n_chunks) % 8:
        n_chunks //= 2
    chunk = per_w // n_chunks
    mesh = plsc.VectorSubcoreMesh(core_axis_name="c", subcore_axis_name="s")

    @functools.partial(
        pl.kernel,
        mesh=mesh,
        compiler_params=pltpu.CompilerParams(use_tc_tiling_on_sc=False),
        out_type=jax.ShapeDtypeStruct((R, D), jnp.float32),
        scratch_types=[
            pltpu.VMEM((chunk,), jnp.int32),
            pltpu.VMEM((chunk, D), jnp.float32),
            pltpu.SemaphoreType.DMA,
        ],
    )
    def k(tbl_hbm, idx_hbm, out_hbm, idx_v, rows_v, sem):
        wid = lax.axis_index("s") * info.num_cores + lax.axis_index("c")
        base = wid * per_w
        for i in range(n_chunks):
            off = base + i * chunk
            pltpu.sync_copy(idx_hbm.at[pl.ds(off, chunk)], idx_v)
            pltpu.async_copy(tbl_hbm.at[idx_v], rows_v, sem).wait()
            pltpu.sync_copy(rows_v, out_hbm.at[pl.ds(off, chunk)])

    return k(table, idx)


def _dot3(ax, ay, az, bx, by, bz):
    return ax * bx + ay * by + az * bz


def _b16(x):
    return x.astype(jnp.bfloat16).astype(jnp.float32)


def _dot3_b16(ax, ay, az, bx, by, bz):
    return (_b16(ax) * _b16(bx) + _b16(ay) * _b16(by)) + _b16(az) * _b16(bz)


def _cross3(ax, ay, az, bx, by, bz):
    return (ay * bz - az * by, az * bx - ax * bz, ax * by - ay * bx)


def _unit_eps(ax, ay, az):
    ln = jnp.sqrt(ax * ax + ay * ay + az * az)
    d = ln + _EPS
    return ax / d, ay / d, az / d, ln


def _feat_kernel(g_ref, c_ref, out_ref, *, nshift):
    g = g_ref[0]
    c = c_ref[0]
    gx, gy, gz = g[0], g[1], g[2]
    gnx, gny, gnz = g[3], g[4], g[5]
    cx, cy, cz = c[0:1], c[1:2], c[2:3]
    ncx, ncy, ncz = c[3:4], c[4:5], c[5:6]

    lx, ly, lz = gx - cx, gy - cy, gz - cz
    dp = _dot3_b16(lx, ly, lz, ncx, ncy, ncz)
    px, py, pz = lx - dp * ncx, ly - dp * ncy, lz - dp * ncz
    plen = jnp.sqrt(px * px + py * py + pz * pz)
    ux, uy, uz = px / plen, py / plen, pz / plen
    ux = jnp.where(jnp.isnan(ux), 0.0, ux)
    uy = jnp.where(jnp.isnan(uy), 0.0, uy)
    uz = jnp.where(jnp.isnan(uz), 0.0, uz)

    kidx = lax.broadcasted_iota(jnp.int32, (_K, _STILE), 0)
    mval = jnp.max(plen, axis=0, keepdims=True)
    ksel = jnp.min(jnp.where(plen == mval, kidx, _K), axis=0, keepdims=True)
    selm = kidx == ksel
    vrx = jnp.sum(jnp.where(selm, ux, 0.0), axis=0, keepdims=True)
    vry = jnp.sum(jnp.where(selm, uy, 0.0), axis=0, keepdims=True)
    vrz = jnp.sum(jnp.where(selm, uz, 0.0), axis=0, keepdims=True)

    dots = _dot3_b16(ux, uy, uz, vrx, vry, vrz)
    crx, cry, crz = _cross3(ux, uy, uz, vrx, vry, vrz)
    sgn = jnp.sign(_dot3_b16(crx, cry, crz, ncx, ncy, ncz))
    sgn = jnp.where(kidx == 0, 1.0, sgn)
    d = sgn * dots - (1.0 - sgn)

    rank = jnp.zeros((_K, _STILE), jnp.int32)
    for j in range(_K):
        dj = d[j : j + 1]
        beats = (dj > d) | ((dj == d) & (j < kidx))
        rank = rank + beats.astype(jnp.int32)

    srt = [jnp.zeros((_K, _STILE), jnp.float32) for _ in range(6)]
    comps = (lx, ly, lz, gnx, gny, gnz)
    for j in range(_K):
        m = rank[j : j + 1] == kidx
        for t in range(6):
            srt[t] = jnp.where(m, comps[t][j : j + 1], srt[t])
    xix, xiy, xiz, xinx, xiny, xinz = srt

    def roll2(a, sh):
        return jnp.concatenate([a[-sh:], a[:-sh]], axis=0)

    ns = nshift
    x3x, x3y, x3z = roll2(xix, ns), roll2(xiy, ns), roll2(xiz, ns)
    x3nx, x3ny, x3nz = roll2(xinx, ns), roll2(xiny, ns), roll2(xinz, ns)
    x4x, x4y, x4z = roll2(xix, -ns), roll2(xiy, -ns), roll2(xiz, -ns)
    x4nx, x4ny, x4nz = roll2(xinx, -ns), roll2(xiny, -ns), roll2(xinz, -ns)

    uax, uay, uaz, l0 = _unit_eps(-xix, -xiy, -xiz)
    s10 = -_dot3(uax, uay, uaz, ncx, ncy, ncz)
    s20 = _dot3(uax, uay, uaz, xinx, xiny, xinz)
    ubx, uby, ubz, l1 = _unit_eps(-x3x, -x3y, -x3z)
    s11 = -_dot3(ubx, uby, ubz, ncx, ncy, ncz)
    s21 = _dot3(ubx, uby, ubz, x3nx, x3ny, x3nz)
    u12x, u12y, u12z, _ = _unit_eps(xix - x3x, xiy - x3y, xiz - x3z)
    s12 = -_dot3(u12x, u12y, u12z, xinx, xiny, xinz)
    s22 = _dot3(u12x, u12y, u12z, x3nx, x3ny, x3nz)
    angle_0 = _dot3(uax, uay, uaz, ubx, uby, ubz)
    angle_1 = _dot3(-ubx, -uby, -ubz, -u12x, -u12y, -u12z)

    pxx, pxy, pxz, _ = _unit_eps(x4x, x4y, x4z)
    p2x, p2y, p2z, _ = _unit_eps(xix, xiy, xiz)
    xxx, xxy, xxz, _ = _unit_eps(x4x - xix, x4y - xiy, x4z - xiz)
    p3x, p3y, p3z, _ = _unit_eps(x3x, x3y, x3z)
    sn1x, sn1y, sn1z = _cross3(pxx, pxy, pxz, p2x, p2y, p2z)
    sn2x, sn2y, sn2z = _cross3(p3x, p3y, p3z, p2x, p2y, p2z)
    a11 = _dot3(pxx, pxy, pxz, p2x, p2y, p2z)
    a12 = _dot3(pxx, pxy, pxz, xxx, xxy, xxz)
    a3 = _dot3(sn1x, sn1y, sn1z, sn2x, sn2y, sn2z)
    a21 = _dot3(xxx, xxy, xxz, x4nx, x4ny, x4nz)
    a22 = _dot3(p2x, p2y, p2z, x4nx, x4ny, x4nz)

    out_ref[0] = jnp.stack(
        [l0, s10, s20, angle_0, s11, s21, angle_1, s12, s22,
         a11, a12, a21, a22, a3],
        axis=0,
    )


def _mlp_kernel(rf_ref, w1t_ref, b1_ref, w2t_ref, b2_ref, out_ref):
    w1t = w1t_ref[...]
    w2t = w2t_ref[...]
    b1 = b1_ref[...]
    b2 = b2_ref[...]
    rf = rf_ref[0]
    for k in range(_K):
        x = rf[:, k, :]
        h = jnp.maximum(
            jnp.dot(w1t, x, preferred_element_type=jnp.float32) + b1, 0.0)
        h = jnp.maximum(
            jnp.dot(w2t, h, preferred_element_type=jnp.float32) + b2, 0.0)
        if k == 0:
            out_ref[0] = h
        else:
            out_ref[0] = jnp.maximum(out_ref[0], h)


def kernel(xyz, norm, fps_idx, knn_idx, W1, b1, W2, b2):
    B, N, _ = xyz.shape
    S = fps_idx.shape[1]
    K = knn_idx.shape[2]

    table = jnp.concatenate(
        [xyz, norm, jnp.zeros((B, N, 2), jnp.float32)], axis=-1
    ).reshape(B * N, 8)
    offs = (jnp.arange(B, dtype=jnp.int32) * N)
    idx_knn = (knn_idx.astype(jnp.int32) + offs[:, None, None]).reshape(-1)
    idx_fps = (fps_idx.astype(jnp.int32) + offs[:, None]).reshape(-1)
    idx_all = jnp.concatenate([idx_knn, idx_fps])
    rows = _sc_gather(table, idx_all)
    grouped = rows[: B * S * K].reshape(B, S, K, 8)
    centres = rows[B * S * K :].reshape(B, S, 8)
    new_xyz = centres[..., 0:3]
    new_norm = centres[..., 3:6]

    G = grouped.transpose(0, 3, 2, 1)
    C = centres.transpose(0, 2, 1)
    rf = pl.pallas_call(
        functools.partial(_feat_kernel, nshift=2 if S >= 1024 else 1),
        grid=(B, S // _STILE),
        in_specs=[
            pl.BlockSpec((1, 8, K, _STILE), lambda b, s: (b, 0, 0, s)),
            pl.BlockSpec((1, 8, _STILE), lambda b, s: (b, 0, s)),
        ],
        out_specs=pl.BlockSpec((1, 14, K, _STILE), lambda b, s: (b, 0, 0, s)),
        out_shape=jax.ShapeDtypeStruct((B, 14, K, S), jnp.float32),
    )(G, C)

    pts_t = pl.pallas_call(
        _mlp_kernel,
        grid=(B,),
        in_specs=[
            pl.BlockSpec((1, 14, K, S), lambda b: (b, 0, 0, 0)),
            pl.BlockSpec((32, 14), lambda b: (0, 0)),
            pl.BlockSpec((32, 1), lambda b: (0, 0)),
            pl.BlockSpec((64, 32), lambda b: (0, 0)),
            pl.BlockSpec((64, 1), lambda b: (0, 0)),
        ],
        out_specs=pl.BlockSpec((1, 64, S), lambda b: (b, 0, 0)),
        out_shape=jax.ShapeDtypeStruct((B, 64, S), jnp.float32),
    )(rf, W1.T, b1.reshape(32, 1), W2.T, b2.reshape(64, 1))
    new_points = pts_t.transpose(0, 2, 1)

    return new_xyz, new_norm, new_points


# speedup vs baseline: 102.1047x; 2.0936x over previous
import functools

import jax
import jax.numpy as jnp
from jax import lax
from jax.experimental import pallas as pl
from jax.experimental.pallas import tpu as pltpu
from jax.experimental.pallas import tpu_sc as plsc

_K = 32
_STILE = 128
_EPS = 1e-07


def _sc_gather(tables, idx):
    R = idx.shape[0]
    info = plsc.get_sparse_core_info()
    nw = info.num_cores * info.num_subcores
    per_w = R // nw
    n_chunks = 8
    while per_w % n_chunks or (per_w // n_chunks) % 8:
        n_chunks //= 2
    chunk = per_w // n_chunks
    mesh = plsc.VectorSubcoreMesh(core_axis_name="c", subcore_axis_name="s")

    @functools.partial(
        pl.kernel,
        mesh=mesh,
        compiler_params=pltpu.CompilerParams(use_tc_tiling_on_sc=False),
        out_type=jax.ShapeDtypeStruct((6, R), jnp.float32),
        scratch_types=[
            pltpu.VMEM((chunk,), jnp.int32),
            pltpu.VMEM((6, chunk), jnp.float32),
            pltpu.SemaphoreType.DMA,
        ],
    )
    def k(tbl_hbm, idx_hbm, out_hbm, idx_v, rows_v, sem):
        wid = lax.axis_index("s") * info.num_cores + lax.axis_index("c")
        base = wid * per_w
        for i in range(n_chunks):
            off = base + i * chunk
            pltpu.sync_copy(idx_hbm.at[pl.ds(off, chunk)], idx_v)
            copies = [
                pltpu.async_copy(tbl_hbm.at[c].at[idx_v], rows_v.at[c], sem)
                for c in range(6)
            ]
            for cp in copies:
                cp.wait()
            pltpu.sync_copy(rows_v, out_hbm.at[:, pl.ds(off, chunk)])

    return k(tables, idx)


def _dot3(ax, ay, az, bx, by, bz):
    return ax * bx + ay * by + az * bz


def _b16(x):
    return x.astype(jnp.bfloat16).astype(jnp.float32)


def _dot3_b16(ax, ay, az, bx, by, bz):
    return (_b16(ax) * _b16(bx) + _b16(ay) * _b16(by)) + _b16(az) * _b16(bz)


def _cross3(ax, ay, az, bx, by, bz):
    return (ay * bz - az * by, az * bx - ax * bz, ax * by - ay * bx)


def _unit_eps(ax, ay, az):
    ln = jnp.sqrt(ax * ax + ay * ay + az * az)
    d = ln + _EPS
    return ax / d, ay / d, az / d, ln


def _feat_kernel(g_ref, c_ref, out_ref, *, nshift):
    g = g_ref[:, 0]
    c = c_ref[:, 0, 0]
    gx, gy, gz = g[0], g[1], g[2]
    gnx, gny, gnz = g[3], g[4], g[5]
    cx, cy, cz = c[0:1], c[1:2], c[2:3]
    ncx, ncy, ncz = c[3:4], c[4:5], c[5:6]

    lx, ly, lz = gx - cx, gy - cy, gz - cz
    dp = _dot3_b16(lx, ly, lz, ncx, ncy, ncz)
    px, py, pz = lx - dp * ncx, ly - dp * ncy, lz - dp * ncz
    plen = jnp.sqrt(px * px + py * py + pz * pz)
    ux, uy, uz = px / plen, py / plen, pz / plen
    ux = jnp.where(jnp.isnan(ux), 0.0, ux)
    uy = jnp.where(jnp.isnan(uy), 0.0, uy)
    uz = jnp.where(jnp.isnan(uz), 0.0, uz)

    kidx = lax.broadcasted_iota(jnp.int32, (_K, _STILE), 0)
    mval = jnp.max(plen, axis=0, keepdims=True)
    ksel = jnp.min(jnp.where(plen == mval, kidx, _K), axis=0, keepdims=True)
    selm = kidx == ksel
    vrx = jnp.sum(jnp.where(selm, ux, 0.0), axis=0, keepdims=True)
    vry = jnp.sum(jnp.where(selm, uy, 0.0), axis=0, keepdims=True)
    vrz = jnp.sum(jnp.where(selm, uz, 0.0), axis=0, keepdims=True)

    dots = _dot3_b16(ux, uy, uz, vrx, vry, vrz)
    crx, cry, crz = _cross3(ux, uy, uz, vrx, vry, vrz)
    sgn = jnp.sign(_dot3_b16(crx, cry, crz, ncx, ncy, ncz))
    sgn = jnp.where(kidx == 0, 1.0, sgn)
    d = sgn * dots - (1.0 - sgn)

    rank = jnp.zeros((_K, _STILE), jnp.int32)
    for j in range(_K):
        dj = d[j : j + 1]
        beats = (dj > d) | ((dj == d) & (j < kidx))
        rank = rank + beats.astype(jnp.int32)

    srt = [jnp.zeros((_K, _STILE), jnp.float32) for _ in range(6)]
    comps = (lx, ly, lz, gnx, gny, gnz)
    for j in range(_K):
        m = rank[j : j + 1] == kidx
        for t in range(6):
            srt[t] = jnp.where(m, comps[t][j : j + 1], srt[t])
    xix, xiy, xiz, xinx, xiny, xinz = srt

    def roll2(a, sh):
        return jnp.concatenate([a[-sh:], a[:-sh]], axis=0)

    ns = nshift
    x3x, x3y, x3z = roll2(xix, ns), roll2(xiy, ns), roll2(xiz, ns)
    x3nx, x3ny, x3nz = roll2(xinx, ns), roll2(xiny, ns), roll2(xinz, ns)
    x4x, x4y, x4z = roll2(xix, -ns), roll2(xiy, -ns), roll2(xiz, -ns)
    x4nx, x4ny, x4nz = roll2(xinx, -ns), roll2(xiny, -ns), roll2(xinz, -ns)

    uax, uay, uaz, l0 = _unit_eps(-xix, -xiy, -xiz)
    s10 = -_dot3(uax, uay, uaz, ncx, ncy, ncz)
    s20 = _dot3(uax, uay, uaz, xinx, xiny, xinz)
    ubx, uby, ubz, l1 = _unit_eps(-x3x, -x3y, -x3z)
    s11 = -_dot3(ubx, uby, ubz, ncx, ncy, ncz)
    s21 = _dot3(ubx, uby, ubz, x3nx, x3ny, x3nz)
    u12x, u12y, u12z, _ = _unit_eps(xix - x3x, xiy - x3y, xiz - x3z)
    s12 = -_dot3(u12x, u12y, u12z, xinx, xiny, xinz)
    s22 = _dot3(u12x, u12y, u12z, x3nx, x3ny, x3nz)
    angle_0 = _dot3(uax, uay, uaz, ubx, uby, ubz)
    angle_1 = _dot3(-ubx, -uby, -ubz, -u12x, -u12y, -u12z)

    pxx, pxy, pxz, _ = _unit_eps(x4x, x4y, x4z)
    p2x, p2y, p2z, _ = _unit_eps(xix, xiy, xiz)
    xxx, xxy, xxz, _ = _unit_eps(x4x - xix, x4y - xiy, x4z - xiz)
    p3x, p3y, p3z, _ = _unit_eps(x3x, x3y, x3z)
    sn1x, sn1y, sn1z = _cross3(pxx, pxy, pxz, p2x, p2y, p2z)
    sn2x, sn2y, sn2z = _cross3(p3x, p3y, p3z, p2x, p2y, p2z)
    a11 = _dot3(pxx, pxy, pxz, p2x, p2y, p2z)
    a12 = _dot3(pxx, pxy, pxz, xxx, xxy, xxz)
    a3 = _dot3(sn1x, sn1y, sn1z, sn2x, sn2y, sn2z)
    a21 = _dot3(xxx, xxy, xxz, x4nx, x4ny, x4nz)
    a22 = _dot3(p2x, p2y, p2z, x4nx, x4ny, x4nz)

    out_ref[0] = jnp.stack(
        [l0, s10, s20, angle_0, s11, s21, angle_1, s12, s22,
         a11, a12, a21, a22, a3],
        axis=0,
    )


def _mlp_kernel(rf_ref, w1t_ref, b1_ref, w2t_ref, b2_ref, out_ref):
    w1t = w1t_ref[...]
    w2t = w2t_ref[...]
    b1 = b1_ref[...]
    b2 = b2_ref[...]
    rf = rf_ref[0]
    for k in range(_K):
        x = rf[:, k, :]
        h = jnp.maximum(
            jnp.dot(w1t, x, preferred_element_type=jnp.float32) + b1, 0.0)
        h = jnp.maximum(
            jnp.dot(w2t, h, preferred_element_type=jnp.float32) + b2, 0.0)
        if k == 0:
            out_ref[0] = h
        else:
            out_ref[0] = jnp.maximum(out_ref[0], h)


def kernel(xyz, norm, fps_idx, knn_idx, W1, b1, W2, b2):
    B, N, _ = xyz.shape
    S = fps_idx.shape[1]
    K = knn_idx.shape[2]

    tables = jnp.concatenate([xyz, norm], axis=-1).transpose(2, 0, 1).reshape(6, B * N)
    offs = (jnp.arange(B, dtype=jnp.int32) * N)
    idx_knn = (knn_idx.astype(jnp.int32).transpose(0, 2, 1)
               + offs[:, None, None]).reshape(-1)
    idx_fps = (fps_idx.astype(jnp.int32) + offs[:, None]).reshape(-1)
    idx_all = jnp.concatenate([idx_knn, idx_fps])
    rows = _sc_gather(tables, idx_all)
    G = rows[:, : B * K * S].reshape(6, B, K, S)
    Cf = rows[:, B * K * S :].reshape(6, B, S)
    new_xyz = Cf[:3].transpose(1, 2, 0)
    new_norm = Cf[3:6].transpose(1, 2, 0)

    rf = pl.pallas_call(
        functools.partial(_feat_kernel, nshift=2 if S >= 1024 else 1),
        grid=(B, S // _STILE),
        in_specs=[
            pl.BlockSpec((6, 1, K, _STILE), lambda b, s: (0, b, 0, s)),
            pl.BlockSpec((6, 1, 1, _STILE), lambda b, s: (0, b, 0, s)),
        ],
        out_specs=pl.BlockSpec((1, 14, K, _STILE), lambda b, s: (b, 0, 0, s)),
        out_shape=jax.ShapeDtypeStruct((B, 14, K, S), jnp.float32),
    )(G, Cf.reshape(6, B, 1, S))

    pts_t = pl.pallas_call(
        _mlp_kernel,
        grid=(B,),
        in_specs=[
            pl.BlockSpec((1, 14, K, S), lambda b: (b, 0, 0, 0)),
            pl.BlockSpec((32, 14), lambda b: (0, 0)),
            pl.BlockSpec((32, 1), lambda b: (0, 0)),
            pl.BlockSpec((64, 32), lambda b: (0, 0)),
            pl.BlockSpec((64, 1), lambda b: (0, 0)),
        ],
        out_specs=pl.BlockSpec((1, 64, S), lambda b: (b, 0, 0)),
        out_shape=jax.ShapeDtypeStruct((B, 64, S), jnp.float32),
    )(rf, W1.T, b1.reshape(32, 1), W2.T, b2.reshape(64, 1))
    new_points = pts_t.transpose(0, 2, 1)

    return new_xyz, new_norm, new_points


# speedup vs baseline: 103.0759x; 1.0095x over previous
import functools

import jax
import jax.numpy as jnp
from jax import lax
from jax.experimental import pallas as pl
from jax.experimental.pallas import tpu as pltpu
from jax.experimental.pallas import tpu_sc as plsc

_K = 32
_STILE = 128
_EPS = 1e-07


def _sc_gather(tables, idx):
    R = idx.shape[0]
    info = plsc.get_sparse_core_info()
    nw = info.num_cores * info.num_subcores
    per_w = R // nw
    n_chunks = 2
    while per_w % n_chunks or (per_w // n_chunks) % 8:
        n_chunks //= 2
    chunk = per_w // n_chunks
    mesh = plsc.VectorSubcoreMesh(core_axis_name="c", subcore_axis_name="s")

    @functools.partial(
        pl.kernel,
        mesh=mesh,
        compiler_params=pltpu.CompilerParams(use_tc_tiling_on_sc=False),
        out_type=jax.ShapeDtypeStruct((6, R), jnp.float32),
        scratch_types=[
            pltpu.VMEM((chunk,), jnp.int32),
            pltpu.VMEM((6, chunk), jnp.float32),
            pltpu.SemaphoreType.DMA,
        ],
    )
    def k(tbl_hbm, idx_hbm, out_hbm, idx_v, rows_v, sem):
        wid = lax.axis_index("s") * info.num_cores + lax.axis_index("c")
        base = wid * per_w
        for i in range(n_chunks):
            off = base + i * chunk
            pltpu.sync_copy(idx_hbm.at[pl.ds(off, chunk)], idx_v)
            copies = [
                pltpu.async_copy(tbl_hbm.at[c].at[idx_v], rows_v.at[c], sem)
                for c in range(6)
            ]
            for cp in copies:
                cp.wait()
            pltpu.sync_copy(rows_v, out_hbm.at[:, pl.ds(off, chunk)])

    return k(tables, idx)


def _dot3(ax, ay, az, bx, by, bz):
    return ax * bx + ay * by + az * bz


def _b16(x):
    return x.astype(jnp.bfloat16).astype(jnp.float32)


def _dot3_b16(ax, ay, az, bx, by, bz):
    return (_b16(ax) * _b16(bx) + _b16(ay) * _b16(by)) + _b16(az) * _b16(bz)


def _cross3(ax, ay, az, bx, by, bz):
    return (ay * bz - az * by, az * bx - ax * bz, ax * by - ay * bx)


def _unit_eps(ax, ay, az):
    ln = jnp.sqrt(ax * ax + ay * ay + az * az)
    d = ln + _EPS
    return ax / d, ay / d, az / d, ln


def _feat_kernel(g_ref, c_ref, out_ref, *, nshift):
    g = g_ref[:, 0]
    c = c_ref[:, 0, 0]
    gx, gy, gz = g[0], g[1], g[2]
    gnx, gny, gnz = g[3], g[4], g[5]
    cx, cy, cz = c[0:1], c[1:2], c[2:3]
    ncx, ncy, ncz = c[3:4], c[4:5], c[5:6]

    lx, ly, lz = gx - cx, gy - cy, gz - cz
    dp = _dot3_b16(lx, ly, lz, ncx, ncy, ncz)
    px, py, pz = lx - dp * ncx, ly - dp * ncy, lz - dp * ncz
    plen = jnp.sqrt(px * px + py * py + pz * pz)
    ux, uy, uz = px / plen, py / plen, pz / plen
    ux = jnp.where(jnp.isnan(ux), 0.0, ux)
    uy = jnp.where(jnp.isnan(uy), 0.0, uy)
    uz = jnp.where(jnp.isnan(uz), 0.0, uz)

    kidx = lax.broadcasted_iota(jnp.int32, (_K, _STILE), 0)
    mval = jnp.max(plen, axis=0, keepdims=True)
    ksel = jnp.min(jnp.where(plen == mval, kidx, _K), axis=0, keepdims=True)
    selm = kidx == ksel
    vrx = jnp.sum(jnp.where(selm, ux, 0.0), axis=0, keepdims=True)
    vry = jnp.sum(jnp.where(selm, uy, 0.0), axis=0, keepdims=True)
    vrz = jnp.sum(jnp.where(selm, uz, 0.0), axis=0, keepdims=True)

    dots = _dot3_b16(ux, uy, uz, vrx, vry, vrz)
    crx, cry, crz = _cross3(ux, uy, uz, vrx, vry, vrz)
    sgn = jnp.sign(_dot3_b16(crx, cry, crz, ncx, ncy, ncz))
    sgn = jnp.where(kidx == 0, 1.0, sgn)
    d = sgn * dots - (1.0 - sgn)

    rank = jnp.zeros((_K, _STILE), jnp.int32)
    for j in range(_K):
        dj = d[j : j + 1]
        beats = (dj > d) | ((dj == d) & (j < kidx))
        rank = rank + beats.astype(jnp.int32)

    srt = [jnp.zeros((_K, _STILE), jnp.float32) for _ in range(6)]
    comps = (lx, ly, lz, gnx, gny, gnz)
    for j in range(_K):
        m = rank[j : j + 1] == kidx
        for t in range(6):
            srt[t] = jnp.where(m, comps[t][j : j + 1], srt[t])
    xix, xiy, xiz, xinx, xiny, xinz = srt

    def roll2(a, sh):
        return jnp.concatenate([a[-sh:], a[:-sh]], axis=0)

    ns = nshift
    x3x, x3y, x3z = roll2(xix, ns), roll2(xiy, ns), roll2(xiz, ns)
    x3nx, x3ny, x3nz = roll2(xinx, ns), roll2(xiny, ns), roll2(xinz, ns)
    x4x, x4y, x4z = roll2(xix, -ns), roll2(xiy, -ns), roll2(xiz, -ns)
    x4nx, x4ny, x4nz = roll2(xinx, -ns), roll2(xiny, -ns), roll2(xinz, -ns)

    uax, uay, uaz, l0 = _unit_eps(-xix, -xiy, -xiz)
    s10 = -_dot3(uax, uay, uaz, ncx, ncy, ncz)
    s20 = _dot3(uax, uay, uaz, xinx, xiny, xinz)
    ubx, uby, ubz, l1 = _unit_eps(-x3x, -x3y, -x3z)
    s11 = -_dot3(ubx, uby, ubz, ncx, ncy, ncz)
    s21 = _dot3(ubx, uby, ubz, x3nx, x3ny, x3nz)
    u12x, u12y, u12z, _ = _unit_eps(xix - x3x, xiy - x3y, xiz - x3z)
    s12 = -_dot3(u12x, u12y, u12z, xinx, xiny, xinz)
    s22 = _dot3(u12x, u12y, u12z, x3nx, x3ny, x3nz)
    angle_0 = _dot3(uax, uay, uaz, ubx, uby, ubz)
    angle_1 = _dot3(-ubx, -uby, -ubz, -u12x, -u12y, -u12z)

    pxx, pxy, pxz, _ = _unit_eps(x4x, x4y, x4z)
    p2x, p2y, p2z, _ = _unit_eps(xix, xiy, xiz)
    xxx, xxy, xxz, _ = _unit_eps(x4x - xix, x4y - xiy, x4z - xiz)
    p3x, p3y, p3z, _ = _unit_eps(x3x, x3y, x3z)
    sn1x, sn1y, sn1z = _cross3(pxx, pxy, pxz, p2x, p2y, p2z)
    sn2x, sn2y, sn2z = _cross3(p3x, p3y, p3z, p2x, p2y, p2z)
    a11 = _dot3(pxx, pxy, pxz, p2x, p2y, p2z)
    a12 = _dot3(pxx, pxy, pxz, xxx, xxy, xxz)
    a3 = _dot3(sn1x, sn1y, sn1z, sn2x, sn2y, sn2z)
    a21 = _dot3(xxx, xxy, xxz, x4nx, x4ny, x4nz)
    a22 = _dot3(p2x, p2y, p2z, x4nx, x4ny, x4nz)

    out_ref[0] = jnp.stack(
        [l0, s10, s20, angle_0, s11, s21, angle_1, s12, s22,
         a11, a12, a21, a22, a3],
        axis=0,
    )


def _mlp_kernel(rf_ref, w1t_ref, b1_ref, w2t_ref, b2_ref, out_ref):
    w1t = w1t_ref[...]
    w2t = w2t_ref[...]
    b1 = b1_ref[...]
    b2 = b2_ref[...]
    rf = rf_ref[0]
    for k in range(_K):
        x = rf[:, k, :]
        h = jnp.maximum(
            jnp.dot(w1t, x, preferred_element_type=jnp.float32) + b1, 0.0)
        h = jnp.maximum(
            jnp.dot(w2t, h, preferred_element_type=jnp.float32) + b2, 0.0)
        if k == 0:
            out_ref[0] = h
        else:
            out_ref[0] = jnp.maximum(out_ref[0], h)


def kernel(xyz, norm, fps_idx, knn_idx, W1, b1, W2, b2):
    B, N, _ = xyz.shape
    S = fps_idx.shape[1]
    K = knn_idx.shape[2]

    tables = jnp.concatenate([xyz, norm], axis=-1).transpose(2, 0, 1).reshape(6, B * N)
    offs = (jnp.arange(B, dtype=jnp.int32) * N)
    idx_knn = (knn_idx.astype(jnp.int32).transpose(0, 2, 1)
               + offs[:, None, None]).reshape(-1)
    idx_fps = (fps_idx.astype(jnp.int32) + offs[:, None]).reshape(-1)
    idx_all = jnp.concatenate([idx_knn, idx_fps])
    rows = _sc_gather(tables, idx_all)
    G = rows[:, : B * K * S].reshape(6, B, K, S)
    Cf = rows[:, B * K * S :].reshape(6, B, S)
    new_xyz = Cf[:3].transpose(1, 2, 0)
    new_norm = Cf[3:6].transpose(1, 2, 0)

    rf = pl.pallas_call(
        functools.partial(_feat_kernel, nshift=2 if S >= 1024 else 1),
        grid=(B, S // _STILE),
        in_specs=[
            pl.BlockSpec((6, 1, K, _STILE), lambda b, s: (0, b, 0, s)),
            pl.BlockSpec((6, 1, 1, _STILE), lambda b, s: (0, b, 0, s)),
        ],
        out_specs=pl.BlockSpec((1, 14, K, _STILE), lambda b, s: (b, 0, 0, s)),
        out_shape=jax.ShapeDtypeStruct((B, 14, K, S), jnp.float32),
    )(G, Cf.reshape(6, B, 1, S))

    pts_t = pl.pallas_call(
        _mlp_kernel,
        grid=(B,),
        in_specs=[
            pl.BlockSpec((1, 14, K, S), lambda b: (b, 0, 0, 0)),
            pl.BlockSpec((32, 14), lambda b: (0, 0)),
            pl.BlockSpec((32, 1), lambda b: (0, 0)),
            pl.BlockSpec((64, 32), lambda b: (0, 0)),
            pl.BlockSpec((64, 1), lambda b: (0, 0)),
        ],
        out_specs=pl.BlockSpec((1, 64, S), lambda b: (b, 0, 0)),
        out_shape=jax.ShapeDtypeStruct((B, 64, S), jnp.float32),
    )(rf, W1.T, b1.reshape(32, 1), W2.T, b2.reshape(64, 1))
    new_points = pts_t.transpose(0, 2, 1)

    return new_xyz, new_norm, new_points
